# baseline (speedup 1.0000x reference)
import jax
import jax.numpy as jnp
from jax.experimental import pallas as pl
from jax.experimental.pallas import tpu as pltpu

_WN = 512
_EV = 64
_T = 50
_D = 100
_IMG = 512
_NW = 100
_GRP = 4
_NRING = 4
_NBUF = _GRP * _NRING


def _nts_kernel(sp_ref,
                A_ref, X_ref, masks_ref, img_ref,
                Wg1_ref, Wg2_ref, Wh1_ref, bh1_ref, Wh2_ref, bh2_ref,
                Wmu_ref, bmu_ref, Weta_ref, beta_b_ref, Wga_ref, bga_ref,
                Wb_ref, wm_ref, wmr_ref, mW1_ref, mb1_ref, mW2_ref, mb2_ref,
                lam_ref, zmat_hbm, betav_ref, gamma_ref, eta_ref, zev_hbm,
                h_hbm, fm_ref, buf_ref, sem, hs_ref, zs_ref, sem2):

    def dot(a, b):
        return jax.lax.dot_general(
            a, b, (((1,), (0,)), ((), ())),
            precision=jax.lax.Precision.DEFAULT,
            preferred_element_type=jnp.float32)

    A = A_ref[...]
    X = X_ref[...]
    H1 = jnp.maximum(dot(A, dot(X, Wg1_ref[...])), 0.0)
    H = dot(A, dot(H1, Wg2_ref[...]))
    hs_ref[...] = H
    pltpu.make_async_copy(hs_ref, h_hbm, sem2.at[0]).start()

    He1 = jnp.maximum(dot(H, Wh1_ref[...]) + bh1_ref[...], 0.0)
    He = jnp.maximum(dot(He1, Wh2_ref[...]) + bh2_ref[...], 0.0)
    rn = jnp.sqrt(jnp.sum(He * He, axis=1, keepdims=True))
    Hn = He / jnp.maximum(rn, 1e-12)

    G = jax.lax.dot_general(
        Hn, Hn, (((1,), (1,)), ((), ())),
        precision=jax.lax.Precision.DEFAULT,
        preferred_element_type=jnp.float32)
    Zm = jnp.maximum(G, 0.0)
    zs_ref[...] = Zm
    pltpu.make_async_copy(zs_ref, zmat_hbm, sem2.at[1]).start()
    dpair = 2.0 - 2.0 * Zm
    fm = jnp.exp(-(dpair * dpair))
    ri = jax.lax.broadcasted_iota(jnp.int32, (_WN, _WN), 0)
    ci = jax.lax.broadcasted_iota(jnp.int32, (_WN, _WN), 1)
    fm_ref[...] = jnp.where(ri == ci, 0.0, fm)

    riv = jax.lax.broadcasted_iota(jnp.int32, (_WN, 1), 0)
    civ = jax.lax.broadcasted_iota(jnp.int32, (1, _WN), 1)
    fmv = fm_ref[...]

    def event_copy(kg, j):
        return pltpu.make_async_copy(
            buf_ref.at[pl.ds(_GRP * j, _GRP)],
            zev_hbm.at[pl.ds(_GRP * kg, _GRP)],
            sem.at[j])

    def masked_slice(k):
        mrow = masks_ref[pl.ds(k, 1), :]
        Li = jnp.sum(jnp.where(mrow != 0, 1, 0).astype(jnp.int32))
        return jnp.where((riv < Li) & (civ < Li), fmv, 0.0)

    @pl.loop(0, _EV // _GRP)
    def _ev(kg):
        j = jax.lax.rem(kg, _NRING)

        @pl.when(kg >= _NRING)
        def _wait_old():
            event_copy(kg - _NRING, j).wait()

        for t in range(_GRP):
            buf_ref[_GRP * j + t] = masked_slice(_GRP * kg + t)
        event_copy(kg, j).start()

    masksf = masks_ref[...]
    mb = jnp.where(masksf != 0, 1.0, 0.0)
    deg = jnp.maximum(jnp.sum(masksf, axis=1, keepdims=True), 1.0)
    Hp = dot(masksf, H) / deg
    mu = jnp.maximum(dot(Hp, Wmu_ref[...]) + bmu_ref[...], 0.0)
    eta = jnp.maximum(dot(Hp, Weta_ref[...]) + beta_b_ref[...], 0.0)
    gamma = jnp.maximum(dot(Hp, Wga_ref[...]) + bga_ref[...], 0.0)
    eta_ref[...] = eta
    gamma_ref[...] = gamma

    wc = wm_ref[...]
    wr = wmr_ref[...]
    cnt = jnp.sum((wr <= wc).astype(jnp.float32), axis=1, keepdims=True)
    sp = sp_ref[0, 0]
    pos = sp * (_NW - 1.0)
    klo = jnp.floor(pos)
    frac = pos - klo
    big = jnp.float32(1e30)
    slo = jnp.min(jnp.where(cnt >= klo + 1.0, wc, big))
    shi = jnp.min(jnp.where(cnt >= klo + 2.0, wc, big))
    shi = jnp.where(frac > 0.0, shi, slo)
    thr = slo + frac * (shi - slo)

    wpr = jnp.where(wc < thr, 0.0, 1.0) * Wb_ref[...]
    bp = dot(H, wpr)
    nb = jnp.sqrt(jnp.sum(bp * bp))
    bv = bp / jnp.maximum(nb, 1e-12)
    betav_ref[...] = bv
    bev = dot(masksf, bv)

    s = dot(mb, Hn)
    rn2 = jnp.sum(Hn * Hn, axis=1, keepdims=True)
    zz = 0.5 * (jnp.sum(s * s, axis=1, keepdims=True) - dot(mb, rn2))
    Zr = jnp.maximum(zz, 0.0)
    lt = jax.nn.sigmoid(mu + bev + eta * jnp.exp(-gamma * Zr))

    hi = jnp.maximum(dot(img_ref[...], mW1_ref[...]) + mb1_ref[...], 0.0)
    li = dot(hi, mW2_ref[...]) + mb2_ref[...]
    lam_ref[...] = jax.nn.sigmoid(lt + li)

    for r in range(_EV // _GRP - _NRING, _EV // _GRP):
        event_copy(r, r % _NRING).wait()
    pltpu.make_async_copy(hs_ref, h_hbm, sem2.at[0]).wait()
    pltpu.make_async_copy(zs_ref, zmat_hbm, sem2.at[1]).wait()


def kernel(epoch, epochs, train_adj, masks, bows_vec, image_features,
           W_gcn1, W_gcn2, W_h1, b_h1, W_h2, b_h2,
           W_mu2, b_mu2, W_eta2, b_eta2, W_gamma2, b_gamma2,
           W_beta, w_m, mlp_W1, mlp_b1, mlp_W2, mlp_b2):
    f32 = jnp.float32
    sparsity = jnp.asarray((epoch / epochs) * 0.3, f32).reshape(1, 1)
    wmc = w_m.reshape(_NW, 1).astype(f32)
    wmr = w_m.reshape(1, _NW).astype(f32)

    vmem = pl.BlockSpec(memory_space=pltpu.VMEM)
    in_specs = [pl.BlockSpec(memory_space=pltpu.SMEM)] + [vmem] * 23
    out_specs = [
        vmem,
        pl.BlockSpec(memory_space=pl.ANY),
        vmem,
        vmem,
        vmem,
        pl.BlockSpec(memory_space=pl.ANY),
        pl.BlockSpec(memory_space=pl.ANY),
    ]
    out_shape = [
        jax.ShapeDtypeStruct((_EV, 1), f32),
        jax.ShapeDtypeStruct((_WN, _WN), f32),
        jax.ShapeDtypeStruct((_WN, 1), f32),
        jax.ShapeDtypeStruct((_EV, 1), f32),
        jax.ShapeDtypeStruct((_EV, 1), f32),
        jax.ShapeDtypeStruct((_EV, _WN, _WN), f32),
        jax.ShapeDtypeStruct((_WN, _D), f32),
    ]

    outs = pl.pallas_call(
        _nts_kernel,
        in_specs=in_specs,
        out_specs=out_specs,
        out_shape=out_shape,
        scratch_shapes=[
            pltpu.VMEM((_WN, _WN), f32),
            pltpu.VMEM((_NBUF, _WN, _WN), f32),
            pltpu.SemaphoreType.DMA((_NRING,)),
            pltpu.VMEM((_WN, _D), f32),
            pltpu.VMEM((_WN, _WN), f32),
            pltpu.SemaphoreType.DMA((2,)),
        ],
    )(sparsity, train_adj, bows_vec, masks, image_features,
      W_gcn1, W_gcn2, W_h1, b_h1.reshape(1, _D), W_h2, b_h2.reshape(1, _D),
      W_mu2, b_mu2.reshape(1, 1), W_eta2, b_eta2.reshape(1, 1),
      W_gamma2, b_gamma2.reshape(1, 1), W_beta, wmc, wmr,
      mlp_W1, mlp_b1.reshape(1, 128), mlp_W2, mlp_b2.reshape(1, 1))

    lam, zmat, betav, gamma, eta, zev, H = outs
    return (lam, zmat, betav.reshape(_WN), gamma, eta, zev, H)


# speedup vs baseline: 1.0007x; 1.0007x over previous
import jax
import jax.numpy as jnp
from jax.experimental import pallas as pl
from jax.experimental.pallas import tpu as pltpu

_WN = 512
_EV = 64
_T = 50
_D = 100
_IMG = 512
_NW = 100
_GRP = 2
_NRING = 8
_NBUF = _GRP * _NRING


def _nts_kernel(sp_ref,
                A_ref, X_ref, masks_ref, img_ref,
                Wg1_ref, Wg2_ref, Wh1_ref, bh1_ref, Wh2_ref, bh2_ref,
                Wmu_ref, bmu_ref, Weta_ref, beta_b_ref, Wga_ref, bga_ref,
                Wb_ref, wm_ref, wmr_ref, mW1_ref, mb1_ref, mW2_ref, mb2_ref,
                lam_ref, zmat_hbm, betav_ref, gamma_ref, eta_ref, zev_hbm,
                h_hbm, fm_ref, buf_ref, sem, hs_ref, zs_ref, sem2):

    def dot(a, b):
        return jax.lax.dot_general(
            a, b, (((1,), (0,)), ((), ())),
            precision=jax.lax.Precision.DEFAULT,
            preferred_element_type=jnp.float32)

    A = A_ref[...]
    X = X_ref[...]
    H1 = jnp.maximum(dot(A, dot(X, Wg1_ref[...])), 0.0)
    H = dot(A, dot(H1, Wg2_ref[...]))
    hs_ref[...] = H
    pltpu.make_async_copy(hs_ref, h_hbm, sem2.at[0]).start()

    He1 = jnp.maximum(dot(H, Wh1_ref[...]) + bh1_ref[...], 0.0)
    He = jnp.maximum(dot(He1, Wh2_ref[...]) + bh2_ref[...], 0.0)
    rn = jnp.sqrt(jnp.sum(He * He, axis=1, keepdims=True))
    Hn = He / jnp.maximum(rn, 1e-12)

    G = jax.lax.dot_general(
        Hn, Hn, (((1,), (1,)), ((), ())),
        precision=jax.lax.Precision.DEFAULT,
        preferred_element_type=jnp.float32)
    Zm = jnp.maximum(G, 0.0)
    zs_ref[...] = Zm
    pltpu.make_async_copy(zs_ref, zmat_hbm, sem2.at[1]).start()
    dpair = 2.0 - 2.0 * Zm
    fm = jnp.exp(-(dpair * dpair))
    ri = jax.lax.broadcasted_iota(jnp.int32, (_WN, _WN), 0)
    ci = jax.lax.broadcasted_iota(jnp.int32, (_WN, _WN), 1)
    fm_ref[...] = jnp.where(ri == ci, 0.0, fm)

    riv = jax.lax.broadcasted_iota(jnp.int32, (_WN, 1), 0)
    civ = jax.lax.broadcasted_iota(jnp.int32, (1, _WN), 1)
    fmv = fm_ref[...]

    def event_copy(kg, j):
        return pltpu.make_async_copy(
            buf_ref.at[pl.ds(_GRP * j, _GRP)],
            zev_hbm.at[pl.ds(_GRP * kg, _GRP)],
            sem.at[j])

    def masked_slice(k):
        mrow = masks_ref[pl.ds(k, 1), :]
        Li = jnp.sum(jnp.where(mrow != 0, 1, 0).astype(jnp.int32))
        return jnp.where((riv < Li) & (civ < Li), fmv, 0.0)

    @pl.loop(0, _EV // _GRP)
    def _ev(kg):
        j = jax.lax.rem(kg, _NRING)

        @pl.when(kg >= _NRING)
        def _wait_old():
            event_copy(kg - _NRING, j).wait()

        for t in range(_GRP):
            buf_ref[_GRP * j + t] = masked_slice(_GRP * kg + t)
        event_copy(kg, j).start()

    masksf = masks_ref[...]
    mb = jnp.where(masksf != 0, 1.0, 0.0)
    deg = jnp.maximum(jnp.sum(masksf, axis=1, keepdims=True), 1.0)
    Hp = dot(masksf, H) / deg
    mu = jnp.maximum(dot(Hp, Wmu_ref[...]) + bmu_ref[...], 0.0)
    eta = jnp.maximum(dot(Hp, Weta_ref[...]) + beta_b_ref[...], 0.0)
    gamma = jnp.maximum(dot(Hp, Wga_ref[...]) + bga_ref[...], 0.0)
    eta_ref[...] = eta
    gamma_ref[...] = gamma

    wc = wm_ref[...]
    wr = wmr_ref[...]
    cnt = jnp.sum((wr <= wc).astype(jnp.float32), axis=1, keepdims=True)
    sp = sp_ref[0, 0]
    pos = sp * (_NW - 1.0)
    klo = jnp.floor(pos)
    frac = pos - klo
    big = jnp.float32(1e30)
    slo = jnp.min(jnp.where(cnt >= klo + 1.0, wc, big))
    shi = jnp.min(jnp.where(cnt >= klo + 2.0, wc, big))
    shi = jnp.where(frac > 0.0, shi, slo)
    thr = slo + frac * (shi - slo)

    wpr = jnp.where(wc < thr, 0.0, 1.0) * Wb_ref[...]
    bp = dot(H, wpr)
    nb = jnp.sqrt(jnp.sum(bp * bp))
    bv = bp / jnp.maximum(nb, 1e-12)
    betav_ref[...] = bv
    bev = dot(masksf, bv)

    s = dot(mb, Hn)
    rn2 = jnp.sum(Hn * Hn, axis=1, keepdims=True)
    zz = 0.5 * (jnp.sum(s * s, axis=1, keepdims=True) - dot(mb, rn2))
    Zr = jnp.maximum(zz, 0.0)
    lt = jax.nn.sigmoid(mu + bev + eta * jnp.exp(-gamma * Zr))

    hi = jnp.maximum(dot(img_ref[...], mW1_ref[...]) + mb1_ref[...], 0.0)
    li = dot(hi, mW2_ref[...]) + mb2_ref[...]
    lam_ref[...] = jax.nn.sigmoid(lt + li)

    for r in range(_EV // _GRP - _NRING, _EV // _GRP):
        event_copy(r, r % _NRING).wait()
    pltpu.make_async_copy(hs_ref, h_hbm, sem2.at[0]).wait()
    pltpu.make_async_copy(zs_ref, zmat_hbm, sem2.at[1]).wait()


def kernel(epoch, epochs, train_adj, masks, bows_vec, image_features,
           W_gcn1, W_gcn2, W_h1, b_h1, W_h2, b_h2,
           W_mu2, b_mu2, W_eta2, b_eta2, W_gamma2, b_gamma2,
           W_beta, w_m, mlp_W1, mlp_b1, mlp_W2, mlp_b2):
    f32 = jnp.float32
    sparsity = jnp.asarray((epoch / epochs) * 0.3, f32).reshape(1, 1)
    wmc = w_m.reshape(_NW, 1).astype(f32)
    wmr = w_m.reshape(1, _NW).astype(f32)

    vmem = pl.BlockSpec(memory_space=pltpu.VMEM)
    in_specs = [pl.BlockSpec(memory_space=pltpu.SMEM)] + [vmem] * 23
    out_specs = [
        vmem,
        pl.BlockSpec(memory_space=pl.ANY),
        vmem,
        vmem,
        vmem,
        pl.BlockSpec(memory_space=pl.ANY),
        pl.BlockSpec(memory_space=pl.ANY),
    ]
    out_shape = [
        jax.ShapeDtypeStruct((_EV, 1), f32),
        jax.ShapeDtypeStruct((_WN, _WN), f32),
        jax.ShapeDtypeStruct((_WN, 1), f32),
        jax.ShapeDtypeStruct((_EV, 1), f32),
        jax.ShapeDtypeStruct((_EV, 1), f32),
        jax.ShapeDtypeStruct((_EV, _WN, _WN), f32),
        jax.ShapeDtypeStruct((_WN, _D), f32),
    ]

    outs = pl.pallas_call(
        _nts_kernel,
        in_specs=in_specs,
        out_specs=out_specs,
        out_shape=out_shape,
        scratch_shapes=[
            pltpu.VMEM((_WN, _WN), f32),
            pltpu.VMEM((_NBUF, _WN, _WN), f32),
            pltpu.SemaphoreType.DMA((_NRING,)),
            pltpu.VMEM((_WN, _D), f32),
            pltpu.VMEM((_WN, _WN), f32),
            pltpu.SemaphoreType.DMA((2,)),
        ],
    )(sparsity, train_adj, bows_vec, masks, image_features,
      W_gcn1, W_gcn2, W_h1, b_h1.reshape(1, _D), W_h2, b_h2.reshape(1, _D),
      W_mu2, b_mu2.reshape(1, 1), W_eta2, b_eta2.reshape(1, 1),
      W_gamma2, b_gamma2.reshape(1, 1), W_beta, wmc, wmr,
      mlp_W1, mlp_b1.reshape(1, 128), mlp_W2, mlp_b2.reshape(1, 1))

    lam, zmat, betav, gamma, eta, zev, H = outs
    return (lam, zmat, betav.reshape(_WN), gamma, eta, zev, H)


# speedup vs baseline: 1.0149x; 1.0142x over previous
import jax
import jax.numpy as jnp
from jax.experimental import pallas as pl
from jax.experimental.pallas import tpu as pltpu

_WN = 512
_EV = 64
_T = 50
_D = 100
_IMG = 512
_NW = 100
_GRP = 2
_NRING = 4
_NBUF = _GRP * _NRING


def _nts_kernel(sp_ref,
                A_ref, X_ref, masks_ref, img_ref,
                Wg1_ref, Wg2_ref, Wh1_ref, bh1_ref, Wh2_ref, bh2_ref,
                Wmu_ref, bmu_ref, Weta_ref, beta_b_ref, Wga_ref, bga_ref,
                Wb_ref, wm_ref, wmr_ref, mW1_ref, mb1_ref, mW2_ref, mb2_ref,
                lam_ref, zmat_hbm, betav_ref, gamma_ref, eta_ref, zev_hbm,
                h_hbm, fm_ref, buf_ref, sem, hs_ref, zs_ref, sem2):

    def dot(a, b):
        return jax.lax.dot_general(
            a, b, (((1,), (0,)), ((), ())),
            precision=jax.lax.Precision.DEFAULT,
            preferred_element_type=jnp.float32)

    A = A_ref[...]
    X = X_ref[...]
    H1 = jnp.maximum(dot(A, dot(X, Wg1_ref[...])), 0.0)
    H = dot(A, dot(H1, Wg2_ref[...]))
    hs_ref[...] = H
    pltpu.make_async_copy(hs_ref, h_hbm, sem2.at[0]).start()

    He1 = jnp.maximum(dot(H, Wh1_ref[...]) + bh1_ref[...], 0.0)
    He = jnp.maximum(dot(He1, Wh2_ref[...]) + bh2_ref[...], 0.0)
    rn = jnp.sqrt(jnp.sum(He * He, axis=1, keepdims=True))
    Hn = He / jnp.maximum(rn, 1e-12)

    G = jax.lax.dot_general(
        Hn, Hn, (((1,), (1,)), ((), ())),
        precision=jax.lax.Precision.DEFAULT,
        preferred_element_type=jnp.float32)
    Zm = jnp.maximum(G, 0.0)
    zs_ref[...] = Zm
    pltpu.make_async_copy(zs_ref, zmat_hbm, sem2.at[1]).start()
    dpair = 2.0 - 2.0 * Zm
    fm = jnp.exp(-(dpair * dpair))
    ri = jax.lax.broadcasted_iota(jnp.int32, (_WN, _WN), 0)
    ci = jax.lax.broadcasted_iota(jnp.int32, (_WN, _WN), 1)
    fm_ref[...] = jnp.where(ri == ci, 0.0, fm)

    riv = jax.lax.broadcasted_iota(jnp.int32, (_WN, 1), 0)
    civ = jax.lax.broadcasted_iota(jnp.int32, (1, _WN), 1)
    fmv = fm_ref[...]

    def event_copy(kg, j):
        return pltpu.make_async_copy(
            buf_ref.at[pl.ds(_GRP * j, _GRP)],
            zev_hbm.at[pl.ds(_GRP * kg, _GRP)],
            sem.at[j])

    def masked_slice(k):
        mrow = masks_ref[pl.ds(k, 1), :]
        Li = jnp.sum(jnp.where(mrow != 0, 1, 0).astype(jnp.int32))
        return jnp.where((riv < Li) & (civ < Li), fmv, 0.0)

    @pl.loop(0, _EV // _GRP)
    def _ev(kg):
        j = jax.lax.rem(kg, _NRING)

        @pl.when(kg >= _NRING)
        def _wait_old():
            event_copy(kg - _NRING, j).wait()

        for t in range(_GRP):
            buf_ref[_GRP * j + t] = masked_slice(_GRP * kg + t)
        event_copy(kg, j).start()

    masksf = masks_ref[...]
    mb = jnp.where(masksf != 0, 1.0, 0.0)
    deg = jnp.maximum(jnp.sum(masksf, axis=1, keepdims=True), 1.0)
    Hp = dot(masksf, H) / deg
    mu = jnp.maximum(dot(Hp, Wmu_ref[...]) + bmu_ref[...], 0.0)
    eta = jnp.maximum(dot(Hp, Weta_ref[...]) + beta_b_ref[...], 0.0)
    gamma = jnp.maximum(dot(Hp, Wga_ref[...]) + bga_ref[...], 0.0)
    eta_ref[...] = eta
    gamma_ref[...] = gamma

    wc = wm_ref[...]
    wr = wmr_ref[...]
    cnt = jnp.sum((wr <= wc).astype(jnp.float32), axis=1, keepdims=True)
    sp = sp_ref[0, 0]
    pos = sp * (_NW - 1.0)
    klo = jnp.floor(pos)
    frac = pos - klo
    big = jnp.float32(1e30)
    slo = jnp.min(jnp.where(cnt >= klo + 1.0, wc, big))
    shi = jnp.min(jnp.where(cnt >= klo + 2.0, wc, big))
    shi = jnp.where(frac > 0.0, shi, slo)
    thr = slo + frac * (shi - slo)

    wpr = jnp.where(wc < thr, 0.0, 1.0) * Wb_ref[...]
    bp = dot(H, wpr)
    nb = jnp.sqrt(jnp.sum(bp * bp))
    bv = bp / jnp.maximum(nb, 1e-12)
    betav_ref[...] = bv
    bev = dot(masksf, bv)

    s = dot(mb, Hn)
    rn2 = jnp.sum(Hn * Hn, axis=1, keepdims=True)
    zz = 0.5 * (jnp.sum(s * s, axis=1, keepdims=True) - dot(mb, rn2))
    Zr = jnp.maximum(zz, 0.0)
    lt = jax.nn.sigmoid(mu + bev + eta * jnp.exp(-gamma * Zr))

    hi = jnp.maximum(dot(img_ref[...], mW1_ref[...]) + mb1_ref[...], 0.0)
    li = dot(hi, mW2_ref[...]) + mb2_ref[...]
    lam_ref[...] = jax.nn.sigmoid(lt + li)

    for r in range(_EV // _GRP - _NRING, _EV // _GRP):
        event_copy(r, r % _NRING).wait()
    pltpu.make_async_copy(hs_ref, h_hbm, sem2.at[0]).wait()
    pltpu.make_async_copy(zs_ref, zmat_hbm, sem2.at[1]).wait()


def kernel(epoch, epochs, train_adj, masks, bows_vec, image_features,
           W_gcn1, W_gcn2, W_h1, b_h1, W_h2, b_h2,
           W_mu2, b_mu2, W_eta2, b_eta2, W_gamma2, b_gamma2,
           W_beta, w_m, mlp_W1, mlp_b1, mlp_W2, mlp_b2):
    f32 = jnp.float32
    sparsity = jnp.asarray((epoch / epochs) * 0.3, f32).reshape(1, 1)
    wmc = w_m.reshape(_NW, 1).astype(f32)
    wmr = w_m.reshape(1, _NW).astype(f32)

    vmem = pl.BlockSpec(memory_space=pltpu.VMEM)
    in_specs = [pl.BlockSpec(memory_space=pltpu.SMEM)] + [vmem] * 23
    out_specs = [
        vmem,
        pl.BlockSpec(memory_space=pl.ANY),
        vmem,
        vmem,
        vmem,
        pl.BlockSpec(memory_space=pl.ANY),
        pl.BlockSpec(memory_space=pl.ANY),
    ]
    out_shape = [
        jax.ShapeDtypeStruct((_EV, 1), f32),
        jax.ShapeDtypeStruct((_WN, _WN), f32),
        jax.ShapeDtypeStruct((_WN, 1), f32),
        jax.ShapeDtypeStruct((_EV, 1), f32),
        jax.ShapeDtypeStruct((_EV, 1), f32),
        jax.ShapeDtypeStruct((_EV, _WN, _WN), f32),
        jax.ShapeDtypeStruct((_WN, _D), f32),
    ]

    outs = pl.pallas_call(
        _nts_kernel,
        in_specs=in_specs,
        out_specs=out_specs,
        out_shape=out_shape,
        scratch_shapes=[
            pltpu.VMEM((_WN, _WN), f32),
            pltpu.VMEM((_NBUF, _WN, _WN), f32),
            pltpu.SemaphoreType.DMA((_NRING,)),
            pltpu.VMEM((_WN, _D), f32),
            pltpu.VMEM((_WN, _WN), f32),
            pltpu.SemaphoreType.DMA((2,)),
        ],
    )(sparsity, train_adj, bows_vec, masks, image_features,
      W_gcn1, W_gcn2, W_h1, b_h1.reshape(1, _D), W_h2, b_h2.reshape(1, _D),
      W_mu2, b_mu2.reshape(1, 1), W_eta2, b_eta2.reshape(1, 1),
      W_gamma2, b_gamma2.reshape(1, 1), W_beta, wmc, wmr,
      mlp_W1, mlp_b1.reshape(1, 128), mlp_W2, mlp_b2.reshape(1, 1))

    lam, zmat, betav, gamma, eta, zev, H = outs
    return (lam, zmat, betav.reshape(_WN), gamma, eta, zev, H)


# speedup vs baseline: 1.0202x; 1.0053x over previous
import jax
import jax.numpy as jnp
from jax.experimental import pallas as pl
from jax.experimental.pallas import tpu as pltpu

_WN = 512
_EV = 64
_T = 50
_D = 100
_IMG = 512
_NW = 100
_GRP = 2
_NRING = 3
_NBUF = _GRP * _NRING


def _nts_kernel(sp_ref,
                A_ref, X_ref, masks_ref, img_ref,
                Wg1_ref, Wg2_ref, Wh1_ref, bh1_ref, Wh2_ref, bh2_ref,
                Wmu_ref, bmu_ref, Weta_ref, beta_b_ref, Wga_ref, bga_ref,
                Wb_ref, wm_ref, wmr_ref, mW1_ref, mb1_ref, mW2_ref, mb2_ref,
                lam_ref, zmat_hbm, betav_ref, gamma_ref, eta_ref, zev_hbm,
                h_hbm, fm_ref, buf_ref, sem, hs_ref, zs_ref, sem2):

    def dot(a, b):
        return jax.lax.dot_general(
            a, b, (((1,), (0,)), ((), ())),
            precision=jax.lax.Precision.DEFAULT,
            preferred_element_type=jnp.float32)

    A = A_ref[...]
    X = X_ref[...]
    H1 = jnp.maximum(dot(A, dot(X, Wg1_ref[...])), 0.0)
    H = dot(A, dot(H1, Wg2_ref[...]))
    hs_ref[...] = H
    pltpu.make_async_copy(hs_ref, h_hbm, sem2.at[0]).start()

    He1 = jnp.maximum(dot(H, Wh1_ref[...]) + bh1_ref[...], 0.0)
    He = jnp.maximum(dot(He1, Wh2_ref[...]) + bh2_ref[...], 0.0)
    rn = jnp.sqrt(jnp.sum(He * He, axis=1, keepdims=True))
    Hn = He / jnp.maximum(rn, 1e-12)

    G = jax.lax.dot_general(
        Hn, Hn, (((1,), (1,)), ((), ())),
        precision=jax.lax.Precision.DEFAULT,
        preferred_element_type=jnp.float32)
    Zm = jnp.maximum(G, 0.0)
    zs_ref[...] = Zm
    pltpu.make_async_copy(zs_ref, zmat_hbm, sem2.at[1]).start()
    dpair = 2.0 - 2.0 * Zm
    fm = jnp.exp(-(dpair * dpair))
    ri = jax.lax.broadcasted_iota(jnp.int32, (_WN, _WN), 0)
    ci = jax.lax.broadcasted_iota(jnp.int32, (_WN, _WN), 1)
    fm_ref[...] = jnp.where(ri == ci, 0.0, fm)

    riv = jax.lax.broadcasted_iota(jnp.int32, (_WN, 1), 0)
    civ = jax.lax.broadcasted_iota(jnp.int32, (1, _WN), 1)
    fmv = fm_ref[...]

    def event_copy(kg, j):
        return pltpu.make_async_copy(
            buf_ref.at[pl.ds(_GRP * j, _GRP)],
            zev_hbm.at[pl.ds(_GRP * kg, _GRP)],
            sem.at[j])

    def masked_slice(k):
        mrow = masks_ref[pl.ds(k, 1), :]
        Li = jnp.sum(jnp.where(mrow != 0, 1, 0).astype(jnp.int32))
        return jnp.where((riv < Li) & (civ < Li), fmv, 0.0)

    @pl.loop(0, _EV // _GRP)
    def _ev(kg):
        j = jax.lax.rem(kg, _NRING)

        @pl.when(kg >= _NRING)
        def _wait_old():
            event_copy(kg - _NRING, j).wait()

        for t in range(_GRP):
            buf_ref[_GRP * j + t] = masked_slice(_GRP * kg + t)
        event_copy(kg, j).start()

    masksf = masks_ref[...]
    mb = jnp.where(masksf != 0, 1.0, 0.0)
    deg = jnp.maximum(jnp.sum(masksf, axis=1, keepdims=True), 1.0)
    Hp = dot(masksf, H) / deg
    mu = jnp.maximum(dot(Hp, Wmu_ref[...]) + bmu_ref[...], 0.0)
    eta = jnp.maximum(dot(Hp, Weta_ref[...]) + beta_b_ref[...], 0.0)
    gamma = jnp.maximum(dot(Hp, Wga_ref[...]) + bga_ref[...], 0.0)
    eta_ref[...] = eta
    gamma_ref[...] = gamma

    wc = wm_ref[...]
    wr = wmr_ref[...]
    cnt = jnp.sum((wr <= wc).astype(jnp.float32), axis=1, keepdims=True)
    sp = sp_ref[0, 0]
    pos = sp * (_NW - 1.0)
    klo = jnp.floor(pos)
    frac = pos - klo
    big = jnp.float32(1e30)
    slo = jnp.min(jnp.where(cnt >= klo + 1.0, wc, big))
    shi = jnp.min(jnp.where(cnt >= klo + 2.0, wc, big))
    shi = jnp.where(frac > 0.0, shi, slo)
    thr = slo + frac * (shi - slo)

    wpr = jnp.where(wc < thr, 0.0, 1.0) * Wb_ref[...]
    bp = dot(H, wpr)
    nb = jnp.sqrt(jnp.sum(bp * bp))
    bv = bp / jnp.maximum(nb, 1e-12)
    betav_ref[...] = bv
    bev = dot(masksf, bv)

    s = dot(mb, Hn)
    rn2 = jnp.sum(Hn * Hn, axis=1, keepdims=True)
    zz = 0.5 * (jnp.sum(s * s, axis=1, keepdims=True) - dot(mb, rn2))
    Zr = jnp.maximum(zz, 0.0)
    lt = jax.nn.sigmoid(mu + bev + eta * jnp.exp(-gamma * Zr))

    hi = jnp.maximum(dot(img_ref[...], mW1_ref[...]) + mb1_ref[...], 0.0)
    li = dot(hi, mW2_ref[...]) + mb2_ref[...]
    lam_ref[...] = jax.nn.sigmoid(lt + li)

    for r in range(_EV // _GRP - _NRING, _EV // _GRP):
        event_copy(r, r % _NRING).wait()
    pltpu.make_async_copy(hs_ref, h_hbm, sem2.at[0]).wait()
    pltpu.make_async_copy(zs_ref, zmat_hbm, sem2.at[1]).wait()


def kernel(epoch, epochs, train_adj, masks, bows_vec, image_features,
           W_gcn1, W_gcn2, W_h1, b_h1, W_h2, b_h2,
           W_mu2, b_mu2, W_eta2, b_eta2, W_gamma2, b_gamma2,
           W_beta, w_m, mlp_W1, mlp_b1, mlp_W2, mlp_b2):
    f32 = jnp.float32
    sparsity = jnp.asarray((epoch / epochs) * 0.3, f32).reshape(1, 1)
    wmc = w_m.reshape(_NW, 1).astype(f32)
    wmr = w_m.reshape(1, _NW).astype(f32)

    vmem = pl.BlockSpec(memory_space=pltpu.VMEM)
    in_specs = [pl.BlockSpec(memory_space=pltpu.SMEM)] + [vmem] * 23
    out_specs = [
        vmem,
        pl.BlockSpec(memory_space=pl.ANY),
        vmem,
        vmem,
        vmem,
        pl.BlockSpec(memory_space=pl.ANY),
        pl.BlockSpec(memory_space=pl.ANY),
    ]
    out_shape = [
        jax.ShapeDtypeStruct((_EV, 1), f32),
        jax.ShapeDtypeStruct((_WN, _WN), f32),
        jax.ShapeDtypeStruct((_WN, 1), f32),
        jax.ShapeDtypeStruct((_EV, 1), f32),
        jax.ShapeDtypeStruct((_EV, 1), f32),
        jax.ShapeDtypeStruct((_EV, _WN, _WN), f32),
        jax.ShapeDtypeStruct((_WN, _D), f32),
    ]

    outs = pl.pallas_call(
        _nts_kernel,
        in_specs=in_specs,
        out_specs=out_specs,
        out_shape=out_shape,
        scratch_shapes=[
            pltpu.VMEM((_WN, _WN), f32),
            pltpu.VMEM((_NBUF, _WN, _WN), f32),
            pltpu.SemaphoreType.DMA((_NRING,)),
            pltpu.VMEM((_WN, _D), f32),
            pltpu.VMEM((_WN, _WN), f32),
            pltpu.SemaphoreType.DMA((2,)),
        ],
    )(sparsity, train_adj, bows_vec, masks, image_features,
      W_gcn1, W_gcn2, W_h1, b_h1.reshape(1, _D), W_h2, b_h2.reshape(1, _D),
      W_mu2, b_mu2.reshape(1, 1), W_eta2, b_eta2.reshape(1, 1),
      W_gamma2, b_gamma2.reshape(1, 1), W_beta, wmc, wmr,
      mlp_W1, mlp_b1.reshape(1, 128), mlp_W2, mlp_b2.reshape(1, 1))

    lam, zmat, betav, gamma, eta, zev, H = outs
    return (lam, zmat, betav.reshape(_WN), gamma, eta, zev, H)
